# baseline (speedup 1.0000x reference)
import functools

import jax
import jax.numpy as jnp
from jax import lax
from jax.experimental import pallas as pl
from jax.experimental.pallas import tpu as pltpu
from jax.experimental.pallas import tpu_sc as plsc

_B, _C, _H, _W = 16, 256, 32, 32
_K, _D = 8192, 256
_N = _B * _H * _W

_TN = 512
_SEG = 2736
_SEGP = 2816
_NSEG = 3

_NC, _NS = 2, 16
_NW = _NC * _NS
_PER_W = _N // _NW
_GCHUNK = 128
_NBUF = 2


def _argmin_body(xsq_ref, esq_ref, x_ref, e_ref, idx_ref):
    x2 = (x_ref[...] * 2.0).astype(jnp.bfloat16)
    xsq = xsq_ref[...]
    lane = lax.broadcasted_iota(jnp.int32, (_TN, 128), 1).astype(jnp.float32)
    seg_v = []
    seg_i = []
    for j in range(_NSEG):
        e = e_ref[pl.ds(j * _SEGP, _SEGP), :]
        dots2 = lax.dot_general(
            x2, e, (((1,), (1,)), ((), ())),
            preferred_element_type=jnp.float32,
        )
        acc_v = None
        for c in range(_SEGP // 128):
            esq = esq_ref[:, pl.ds(j * _SEGP + c * 128, 128)]
            d_c = (xsq - dots2[:, c * 128:(c + 1) * 128]) + esq
            i_c = lane + jnp.float32(c * 128)
            if acc_v is None:
                acc_v, acc_i = d_c, i_c
            else:
                take = d_c < acc_v
                acc_v = jnp.minimum(acc_v, d_c)
                acc_i = jnp.where(take, i_c, acc_i)
        tmin = jnp.min(acc_v, axis=1, keepdims=True)
        tidx = jnp.min(jnp.where(acc_v == tmin, acc_i, jnp.float32(_K)),
                       axis=1, keepdims=True)
        seg_v.append(tmin)
        seg_i.append(tidx.astype(jnp.int32) + j * _SEG)
    cur_v = seg_v[0].astype(jnp.bfloat16).astype(jnp.float32)
    cur_i = seg_i[0]
    for j in (1, 2):
        take = seg_v[j] < cur_v
        cur_v = jnp.where(take, seg_v[j], cur_v).astype(
            jnp.bfloat16).astype(jnp.float32)
        cur_i = jnp.where(take, seg_i[j], cur_i)
    idx_ref[...] = cur_i


def _argmin_indices(flat, e_pad, xsq, esq_pad):
    return pl.pallas_call(
        _argmin_body,
        grid=(_N // _TN,),
        in_specs=[
            pl.BlockSpec((_TN, 1), lambda i: (i, 0)),
            pl.BlockSpec((1, _NSEG * _SEGP), lambda i: (0, 0)),
            pl.BlockSpec((_TN, _D), lambda i: (i, 0)),
            pl.BlockSpec((_NSEG * _SEGP, _D), lambda i: (0, 0)),
        ],
        out_specs=pl.BlockSpec((_TN, 1), lambda i: (i, 0)),
        out_shape=jax.ShapeDtypeStruct((_N, 1), jnp.int32),
    )(xsq, esq_pad, flat, e_pad)


@functools.cache
def _make_sc_gather():
    mesh = plsc.VectorSubcoreMesh(core_axis_name="c", subcore_axis_name="s")
    nchunk = _PER_W // _GCHUNK

    @functools.partial(
        pl.kernel,
        mesh=mesh,
        out_type=jax.ShapeDtypeStruct((_N, _D), jnp.float32),
        scratch_types=[
            pltpu.VMEM((_PER_W,), jnp.int32),
            pltpu.VMEM((_NBUF, _GCHUNK, _D), jnp.float32),
            pltpu.SemaphoreType.DMA,
            pltpu.SemaphoreType.DMA,
        ],
    )
    def _sc_gather(idx_hbm, table_hbm, out_hbm, idx_v, rows_v, sem0, sem1):
        wid = lax.axis_index("s") * _NC + lax.axis_index("c")
        base = wid * _PER_W
        pltpu.sync_copy(idx_hbm.at[pl.ds(base, _PER_W)], idx_v)
        sems = (sem0, sem1)
        copies = []
        for ci in range(nchunk):
            b = ci % _NBUF
            c = pltpu.async_copy(
                table_hbm.at[idx_v.at[pl.ds(ci * _GCHUNK, _GCHUNK)]],
                rows_v.at[b], sems[b])
            copies.append(c)
            if ci >= _NBUF - 1:
                w = ci - (_NBUF - 1)
                copies[w].wait()
                pltpu.sync_copy(
                    rows_v.at[w % _NBUF],
                    out_hbm.at[pl.ds(base + w * _GCHUNK, _GCHUNK), :])
        for w in range(nchunk - _NBUF + 1, nchunk):
            copies[w].wait()
            pltpu.sync_copy(
                rows_v.at[w % _NBUF],
                out_hbm.at[pl.ds(base + w * _GCHUNK, _GCHUNK), :])

    return _sc_gather


def kernel(x, embed_weight):
    x_p = jnp.transpose(x, (0, 2, 3, 1))
    flat = x_p.reshape(-1, _D)
    xsq = jnp.sum(flat * flat, axis=1, keepdims=True)
    esq = jnp.sum(embed_weight * embed_weight, axis=1)
    eb = embed_weight.astype(jnp.bfloat16)
    segs_e = []
    segs_q = []
    for j in range(_NSEG):
        lo = j * _SEG
        hi = min(lo + _SEG, _K)
        pad = _SEGP - (hi - lo)
        segs_e.append(jnp.pad(eb[lo:hi], ((0, pad), (0, 0))))
        segs_q.append(jnp.pad(esq[lo:hi], (0, pad),
                              constant_values=jnp.inf))
    e_pad = jnp.concatenate(segs_e, axis=0)
    esq_pad = jnp.concatenate(segs_q)[None, :]
    idx = _argmin_indices(flat, e_pad, xsq, esq_pad)
    quantized = _make_sc_gather()(idx.reshape(_N), embed_weight)
    return quantized.reshape(_B, _H, _W, _D)
